# baseline (speedup 1.0000x reference)
import functools

import jax
import jax.numpy as jnp
from jax import lax
from jax.experimental import pallas as pl
from jax.experimental.pallas import tpu as pltpu
from jax.experimental.pallas import tpu_sc as plsc

_F32 = jnp.float32
_HIGHEST = lax.Precision.HIGHEST


def _make_halos(x, th):
    b, h, w, c = x.shape
    t = h // th
    zrow = jnp.zeros((b, 1, w, c), x.dtype)
    top = jnp.concatenate([zrow, x[:, th - 1::th][:, : t - 1]], axis=1)
    bot = jnp.concatenate([x[:, th::th], zrow], axis=1)
    return (top.reshape(b, t, 1, w, c), bot.reshape(b, t, 1, w, c))


def _conv3x3(x, w_oihw, bias, th, act, im2col, out_dtype=_F32,
             prec=lax.Precision.DEFAULT):
    b, h, wd, cin = x.shape
    cout = w_oihw.shape[0]
    t = h // th
    x = x.astype(jnp.bfloat16)
    top, bot = _make_halos(x, th)
    w_hwio = jnp.transpose(w_oihw, (2, 3, 1, 0)).astype(jnp.bfloat16)
    dx_merge = False
    if im2col:
        wk = w_hwio.reshape(9 * cin, cout)
    elif dx_merge:
        wk = jnp.transpose(w_hwio, (0, 2, 1, 3)).reshape(3, cin, 3 * cout)
    else:
        wk = w_hwio.reshape(9, cin, cout)
    b2 = bias.reshape(1, cout)

    def body(x_ref, top_ref, bot_ref, w_ref, b_ref, o_ref):
        win = jnp.concatenate([top_ref[0, 0], x_ref[0], bot_ref[0, 0]], axis=0)
        zc = jnp.zeros((th + 2, 1, cin), win.dtype)
        win = jnp.concatenate([zc, win, zc], axis=1)
        if im2col:
            cols = jnp.concatenate(
                [win[dy:dy + th, dx:dx + wd, :]
                 for dy in range(3) for dx in range(3)], axis=-1)
            res = jnp.dot(cols.reshape(th * wd, 9 * cin), w_ref[...],
                          precision=prec, preferred_element_type=_F32)
        elif dx_merge:
            res = jnp.zeros((th, wd, cout), _F32)
            for dy in range(3):
                p = jnp.dot(win[dy:dy + th].reshape(th * (wd + 2), cin),
                            w_ref[dy], precision=prec,
                            preferred_element_type=_F32)
                p = p.reshape(th, wd + 2, 3 * cout)
                for dx in range(3):
                    res = res + p[:, dx:dx + wd, dx * cout:(dx + 1) * cout]
            res = res.reshape(th * wd, cout)
        else:
            res = jnp.zeros((th * wd, cout), _F32)
            for dy in range(3):
                for dx in range(3):
                    sl = win[dy:dy + th, dx:dx + wd, :].reshape(th * wd, cin)
                    res = res + jnp.dot(sl, w_ref[dy * 3 + dx],
                                        precision=prec,
                                        preferred_element_type=_F32)
        res = act(res + b_ref[0])
        o_ref[0] = res.reshape(th, wd, cout).astype(out_dtype)

    return pl.pallas_call(
        body,
        grid=(b, t),
        in_specs=[
            pl.BlockSpec((1, th, wd, cin), lambda i, j: (i, j, 0, 0)),
            pl.BlockSpec((1, 1, 1, wd, cin), lambda i, j: (i, j, 0, 0, 0)),
            pl.BlockSpec((1, 1, 1, wd, cin), lambda i, j: (i, j, 0, 0, 0)),
            pl.BlockSpec(wk.shape, lambda i, j: (0,) * wk.ndim),
            pl.BlockSpec((1, cout), lambda i, j: (0, 0)),
        ],
        out_specs=pl.BlockSpec((1, th, wd, cout), lambda i, j: (i, j, 0, 0)),
        out_shape=jax.ShapeDtypeStruct((b, h, wd, cout), out_dtype),
        compiler_params=pltpu.CompilerParams(
            dimension_semantics=("arbitrary", "arbitrary"),
            vmem_limit_bytes=100 * 1024 * 1024,
        ),
    )(x, top, bot, wk, b2)


_PB = 1024


def _vq_argmin(enc2d, codebook):
    n, d = enc2d.shape
    k = codebook.shape[0]
    nblk = n // _PB

    def body(x_ref, cb_ref, idx_ref, loss_ref):
        x = x_ref[...]
        cb = cb_ref[...]
        toks = jnp.swapaxes(x.reshape(_PB // 64, 64, 64), 1, 2).reshape(_PB, 64)
        bf = jnp.bfloat16
        dims = (((1,), (1,)), ((), ()))
        t_hi = toks.astype(bf)
        t_lo = (toks - t_hi.astype(_F32)).astype(bf)
        c_hi = cb.astype(bf)
        c_lo = (cb - c_hi.astype(_F32)).astype(bf)
        mm = (lax.dot_general(t_hi, c_hi, dims, preferred_element_type=_F32)
              + (lax.dot_general(t_hi, c_lo, dims, preferred_element_type=_F32)
                 + lax.dot_general(t_lo, c_hi, dims,
                                   preferred_element_type=_F32)))
        xn = jnp.sum(toks * toks, axis=1)[:, None]
        cn = jnp.sum(cb * cb, axis=1)[None, :]
        dist = xn + cn - 2.0 * mm
        m = jnp.min(dist, axis=1, keepdims=True)
        iota = lax.broadcasted_iota(jnp.int32, (_PB, k), 1)
        idx = jnp.min(jnp.where(dist == m, iota, k), axis=1)
        idx_ref[0, 0] = idx

        @pl.when(pl.program_id(0) == 0)
        def _():
            loss_ref[...] = jnp.zeros((1, 1), _F32)

        loss_ref[...] += jnp.sum(m).reshape(1, 1)

    return pl.pallas_call(
        body,
        grid=(nblk,),
        in_specs=[
            pl.BlockSpec((_PB, d), lambda i: (i, 0)),
            pl.BlockSpec((k, d), lambda i: (0, 0)),
        ],
        out_specs=[
            pl.BlockSpec((1, 1, _PB), lambda i: (i, 0, 0)),
            pl.BlockSpec((1, 1), lambda i: (0, 0)),
        ],
        out_shape=[
            jax.ShapeDtypeStruct((nblk, 1, _PB), jnp.int32),
            jax.ShapeDtypeStruct((1, 1), _F32),
        ],
        compiler_params=pltpu.CompilerParams(
            dimension_semantics=("arbitrary",),
            vmem_limit_bytes=100 * 1024 * 1024,
        ),
    )(enc2d, codebook)


_NW = 32
_CHT = 392
_NCHK = 16


def _sc_gather(codebook, idx_flat):
    n = idx_flat.shape[0]
    d = codebook.shape[1]
    kk = codebook.shape[0]
    per_w = n // _NW
    mesh = plsc.VectorSubcoreMesh(core_axis_name="c", subcore_axis_name="s")

    @functools.partial(
        pl.kernel, mesh=mesh,
        compiler_params=pltpu.CompilerParams(use_tc_tiling_on_sc=False),
        out_type=jax.ShapeDtypeStruct((n * d,), jnp.bfloat16),
        scratch_types=[
            pltpu.VMEM((kk * d,), jnp.bfloat16),
            pltpu.VMEM((per_w + 8,), jnp.int32),
            pltpu.VMEM((2, _CHT * d), jnp.bfloat16),
            pltpu.SemaphoreType.DMA,
            pltpu.SemaphoreType.DMA,
            pltpu.SemaphoreType.DMA,
        ],
    )
    def k(cb_hbm, idx_hbm, out_hbm, cb_v, idx_v, stg, lsem, wsem0, wsem1):
        wid = lax.axis_index("s") * 2 + lax.axis_index("c")
        base = wid * per_w
        pltpu.async_copy(cb_hbm, cb_v, lsem)
        pltpu.async_copy(idx_hbm.at[pl.ds(base, per_w)],
                         idx_v.at[pl.ds(0, per_w)], lsem).wait()
        pltpu.make_async_copy(cb_hbm, cb_v, lsem).wait()

        def pair(p, carry):
            for par in range(2):
                wsem = (wsem0, wsem1)[par]
                c = 2 * p + par

                @pl.when(p > 0)
                def _():
                    pltpu.make_async_copy(
                        stg.at[par], out_hbm.at[pl.ds(0, _CHT * d)],
                        wsem).wait()

                def token8(i8, carry2):
                    vec = idx_v[pl.ds(c * _CHT + i8 * 8, 16)]
                    for l in range(8):
                        s = vec[l]
                        t = i8 * 8 + l
                        for q in range(2):
                            stg[par, pl.ds(t * d + q * 32, 32)] = (
                                cb_v[pl.ds(s * d + q * 32, 32)])
                    return carry2

                lax.fori_loop(0, _CHT // 8, token8, 0)
                pltpu.async_copy(
                    stg.at[par],
                    out_hbm.at[pl.ds((base + c * _CHT) * d, _CHT * d)], wsem)
            return carry

        lax.fori_loop(0, _NCHK // 2, pair, 0)
        for par in range(2):
            wsem = (wsem0, wsem1)[par]
            pltpu.make_async_copy(stg.at[par],
                                  out_hbm.at[pl.ds(0, _CHT * d)], wsem).wait()

    return k(codebook.astype(jnp.bfloat16).reshape(-1), idx_flat).reshape(n, d)


def kernel(x, enc_w1, enc_b1, enc_w2, enc_b2, codebook,
           dec_w1, dec_b1, dec_w2, dec_b2):
    b, _, h, w = x.shape
    d = codebook.shape[1]
    relu = lambda v: jnp.maximum(v, 0.0)

    x_nhwc = jnp.transpose(x, (0, 2, 3, 1))
    h1 = _conv3x3(x_nhwc, enc_w1, enc_b1, 16, relu, im2col=True,
                  out_dtype=jnp.bfloat16)
    enc = _conv3x3(h1, enc_w2, enc_b2, 8, relu, im2col=False)

    enc2d = enc.reshape(b * h * w, d)
    idx_blocks, loss_sum = _vq_argmin(enc2d, codebook)
    n_tok = b * h * w
    groups = h * w // d
    idx_flat = (idx_blocks.reshape(b, groups, d)
                .transpose(0, 2, 1).reshape(n_tok))

    quant = _sc_gather(codebook, idx_flat)
    q_nhwc = jnp.transpose(quant.reshape(b, d, h, w), (0, 2, 3, 1))

    d1 = _conv3x3(q_nhwc, dec_w1, dec_b1, 8, relu, im2col=False,
                  out_dtype=jnp.bfloat16)
    out_nhwc = _conv3x3(d1, dec_w2, dec_b2, 8, jax.nn.sigmoid, im2col=False)
    reconstructed = jnp.transpose(out_nhwc, (0, 3, 1, 2))

    loss = loss_sum[0, 0] * (1.25 / (n_tok * d))
    return (reconstructed, loss)


# speedup vs baseline: 1.0328x; 1.0328x over previous
import functools

import jax
import jax.numpy as jnp
from jax import lax
from jax.experimental import pallas as pl
from jax.experimental.pallas import tpu as pltpu
from jax.experimental.pallas import tpu_sc as plsc

_F32 = jnp.float32
_HIGHEST = lax.Precision.HIGHEST


def _make_halos(x, th):
    b, h, w, c = x.shape
    t = h // th
    zrow = jnp.zeros((b, 1, w, c), x.dtype)
    top = jnp.concatenate([zrow, x[:, th - 1::th][:, : t - 1]], axis=1)
    bot = jnp.concatenate([x[:, th::th], zrow], axis=1)
    return (top.reshape(b, t, 1, w, c), bot.reshape(b, t, 1, w, c))


def _conv3x3(x, w_oihw, bias, th, act, im2col, out_dtype=_F32,
             prec=lax.Precision.DEFAULT):
    b, h, wd, cin = x.shape
    cout = w_oihw.shape[0]
    t = h // th
    x = x.astype(jnp.bfloat16)
    top, bot = _make_halos(x, th)
    w_hwio = jnp.transpose(w_oihw, (2, 3, 1, 0)).astype(jnp.bfloat16)
    dx_merge = False
    if im2col:
        wk = w_hwio.reshape(9 * cin, cout)
    elif dx_merge:
        wk = jnp.transpose(w_hwio, (0, 2, 1, 3)).reshape(3, cin, 3 * cout)
    else:
        wk = w_hwio.reshape(9, cin, cout)
    b2 = bias.reshape(1, cout)

    def body(x_ref, top_ref, bot_ref, w_ref, b_ref, o_ref):
        win = jnp.concatenate([top_ref[0, 0], x_ref[0], bot_ref[0, 0]], axis=0)
        zc = jnp.zeros((th + 2, 1, cin), win.dtype)
        win = jnp.concatenate([zc, win, zc], axis=1)
        if im2col:
            cols = jnp.concatenate(
                [win[dy:dy + th, dx:dx + wd, :]
                 for dy in range(3) for dx in range(3)], axis=-1)
            res = jnp.dot(cols.reshape(th * wd, 9 * cin), w_ref[...],
                          precision=prec, preferred_element_type=_F32)
        elif dx_merge:
            res = jnp.zeros((th, wd, cout), _F32)
            for dy in range(3):
                p = jnp.dot(win[dy:dy + th].reshape(th * (wd + 2), cin),
                            w_ref[dy], precision=prec,
                            preferred_element_type=_F32)
                p = p.reshape(th, wd + 2, 3 * cout)
                for dx in range(3):
                    res = res + p[:, dx:dx + wd, dx * cout:(dx + 1) * cout]
            res = res.reshape(th * wd, cout)
        else:
            res = jnp.zeros((th * wd, cout), _F32)
            for dy in range(3):
                for dx in range(3):
                    sl = win[dy:dy + th, dx:dx + wd, :].reshape(th * wd, cin)
                    res = res + jnp.dot(sl, w_ref[dy * 3 + dx],
                                        precision=prec,
                                        preferred_element_type=_F32)
        res = act(res + b_ref[0])
        o_ref[0] = res.reshape(th, wd, cout).astype(out_dtype)

    return pl.pallas_call(
        body,
        grid=(b, t),
        in_specs=[
            pl.BlockSpec((1, th, wd, cin), lambda i, j: (i, j, 0, 0)),
            pl.BlockSpec((1, 1, 1, wd, cin), lambda i, j: (i, j, 0, 0, 0)),
            pl.BlockSpec((1, 1, 1, wd, cin), lambda i, j: (i, j, 0, 0, 0)),
            pl.BlockSpec(wk.shape, lambda i, j: (0,) * wk.ndim),
            pl.BlockSpec((1, cout), lambda i, j: (0, 0)),
        ],
        out_specs=pl.BlockSpec((1, th, wd, cout), lambda i, j: (i, j, 0, 0)),
        out_shape=jax.ShapeDtypeStruct((b, h, wd, cout), out_dtype),
        compiler_params=pltpu.CompilerParams(
            dimension_semantics=("arbitrary", "arbitrary"),
            vmem_limit_bytes=100 * 1024 * 1024,
        ),
    )(x, top, bot, wk, b2)


_PB = 2048


def _vq_argmin(enc2d, codebook):
    n, d = enc2d.shape
    k = codebook.shape[0]
    nblk = n // _PB

    def body(x_ref, cb_ref, idx_ref, loss_ref):
        x = x_ref[...]
        cb = cb_ref[...]
        toks = jnp.swapaxes(x.reshape(_PB // 64, 64, 64), 1, 2).reshape(_PB, 64)
        bf = jnp.bfloat16
        dims = (((1,), (1,)), ((), ()))
        t_hi = toks.astype(bf)
        t_lo = (toks - t_hi.astype(_F32)).astype(bf)
        c_hi = cb.astype(bf)
        c_lo = (cb - c_hi.astype(_F32)).astype(bf)
        mm = (lax.dot_general(t_hi, c_hi, dims, preferred_element_type=_F32)
              + (lax.dot_general(t_hi, c_lo, dims, preferred_element_type=_F32)
                 + lax.dot_general(t_lo, c_hi, dims,
                                   preferred_element_type=_F32)))
        xn = jnp.sum(toks * toks, axis=1)[:, None]
        cn = jnp.sum(cb * cb, axis=1)[None, :]
        dist = xn + cn - 2.0 * mm
        m = jnp.min(dist, axis=1, keepdims=True)
        iota = lax.broadcasted_iota(jnp.int32, (_PB, k), 1)
        idx = jnp.min(jnp.where(dist == m, iota, k), axis=1)
        idx_ref[0, 0] = idx

        @pl.when(pl.program_id(0) == 0)
        def _():
            loss_ref[...] = jnp.zeros((1, 1), _F32)

        loss_ref[...] += jnp.sum(m).reshape(1, 1)

    return pl.pallas_call(
        body,
        grid=(nblk,),
        in_specs=[
            pl.BlockSpec((_PB, d), lambda i: (i, 0)),
            pl.BlockSpec((k, d), lambda i: (0, 0)),
        ],
        out_specs=[
            pl.BlockSpec((1, 1, _PB), lambda i: (i, 0, 0)),
            pl.BlockSpec((1, 1), lambda i: (0, 0)),
        ],
        out_shape=[
            jax.ShapeDtypeStruct((nblk, 1, _PB), jnp.int32),
            jax.ShapeDtypeStruct((1, 1), _F32),
        ],
        compiler_params=pltpu.CompilerParams(
            dimension_semantics=("arbitrary",),
            vmem_limit_bytes=100 * 1024 * 1024,
        ),
    )(enc2d, codebook)


_NW = 32
_CHT = 392
_NCHK = 16


def _sc_gather(codebook, idx_flat):
    n = idx_flat.shape[0]
    d = codebook.shape[1]
    kk = codebook.shape[0]
    per_w = n // _NW
    mesh = plsc.VectorSubcoreMesh(core_axis_name="c", subcore_axis_name="s")

    @functools.partial(
        pl.kernel, mesh=mesh,
        compiler_params=pltpu.CompilerParams(use_tc_tiling_on_sc=False),
        out_type=jax.ShapeDtypeStruct((n * d,), jnp.bfloat16),
        scratch_types=[
            pltpu.VMEM((kk * d,), jnp.bfloat16),
            pltpu.VMEM((per_w + 8,), jnp.int32),
            pltpu.VMEM((2, _CHT * d), jnp.bfloat16),
            pltpu.SemaphoreType.DMA,
            pltpu.SemaphoreType.DMA,
            pltpu.SemaphoreType.DMA,
        ],
    )
    def k(cb_hbm, idx_hbm, out_hbm, cb_v, idx_v, stg, lsem, wsem0, wsem1):
        wid = lax.axis_index("s") * 2 + lax.axis_index("c")
        base = wid * per_w
        pltpu.async_copy(cb_hbm, cb_v, lsem)
        pltpu.async_copy(idx_hbm.at[pl.ds(base, per_w)],
                         idx_v.at[pl.ds(0, per_w)], lsem).wait()
        pltpu.make_async_copy(cb_hbm, cb_v, lsem).wait()

        def pair(p, carry):
            for par in range(2):
                wsem = (wsem0, wsem1)[par]
                c = 2 * p + par

                @pl.when(p > 0)
                def _():
                    pltpu.make_async_copy(
                        stg.at[par], out_hbm.at[pl.ds(0, _CHT * d)],
                        wsem).wait()

                def token8(i8, carry2):
                    vec = idx_v[pl.ds(c * _CHT + i8 * 8, 16)]
                    for l in range(8):
                        s = vec[l]
                        t = i8 * 8 + l
                        for q in range(2):
                            stg[par, pl.ds(t * d + q * 32, 32)] = (
                                cb_v[pl.ds(s * d + q * 32, 32)])
                    return carry2

                lax.fori_loop(0, _CHT // 8, token8, 0)
                pltpu.async_copy(
                    stg.at[par],
                    out_hbm.at[pl.ds((base + c * _CHT) * d, _CHT * d)], wsem)
            return carry

        lax.fori_loop(0, _NCHK // 2, pair, 0)
        for par in range(2):
            wsem = (wsem0, wsem1)[par]
            pltpu.make_async_copy(stg.at[par],
                                  out_hbm.at[pl.ds(0, _CHT * d)], wsem).wait()

    return k(codebook.astype(jnp.bfloat16).reshape(-1), idx_flat).reshape(n, d)


def kernel(x, enc_w1, enc_b1, enc_w2, enc_b2, codebook,
           dec_w1, dec_b1, dec_w2, dec_b2):
    b, _, h, w = x.shape
    d = codebook.shape[1]
    relu = lambda v: jnp.maximum(v, 0.0)

    x_nhwc = jnp.transpose(x, (0, 2, 3, 1))
    h1 = _conv3x3(x_nhwc, enc_w1, enc_b1, 16, relu, im2col=True,
                  out_dtype=jnp.bfloat16)
    enc = _conv3x3(h1, enc_w2, enc_b2, 8, relu, im2col=False)

    enc2d = enc.reshape(b * h * w, d)
    idx_blocks, loss_sum = _vq_argmin(enc2d, codebook)
    n_tok = b * h * w
    groups = h * w // d
    idx_flat = (idx_blocks.reshape(b, groups, d)
                .transpose(0, 2, 1).reshape(n_tok))

    quant = _sc_gather(codebook, idx_flat)
    q_nhwc = jnp.transpose(quant.reshape(b, d, h, w), (0, 2, 3, 1))

    d1 = _conv3x3(q_nhwc, dec_w1, dec_b1, 8, relu, im2col=False,
                  out_dtype=jnp.bfloat16)
    out_nhwc = _conv3x3(d1, dec_w2, dec_b2, 8, jax.nn.sigmoid, im2col=False)
    reconstructed = jnp.transpose(out_nhwc, (0, 3, 1, 2))

    loss = loss_sum[0, 0] * (1.25 / (n_tok * d))
    return (reconstructed, loss)


# speedup vs baseline: 1.0333x; 1.0005x over previous
import functools

import jax
import jax.numpy as jnp
from jax import lax
from jax.experimental import pallas as pl
from jax.experimental.pallas import tpu as pltpu
from jax.experimental.pallas import tpu_sc as plsc

_F32 = jnp.float32
_HIGHEST = lax.Precision.HIGHEST


def _make_halos(x, th):
    b, h, w, c = x.shape
    t = h // th
    zrow = jnp.zeros((b, 1, w, c), x.dtype)
    top = jnp.concatenate([zrow, x[:, th - 1::th][:, : t - 1]], axis=1)
    bot = jnp.concatenate([x[:, th::th], zrow], axis=1)
    return (top.reshape(b, t, 1, w, c), bot.reshape(b, t, 1, w, c))


def _conv3x3(x, w_oihw, bias, th, act, im2col, out_dtype=_F32,
             prec=lax.Precision.DEFAULT):
    b, h, wd, cin = x.shape
    cout = w_oihw.shape[0]
    t = h // th
    x = x.astype(jnp.bfloat16)
    top, bot = _make_halos(x, th)
    w_hwio = jnp.transpose(w_oihw, (2, 3, 1, 0)).astype(jnp.bfloat16)
    if im2col:
        wk = w_hwio.reshape(9 * cin, cout)
    else:
        wk = w_hwio.reshape(9, cin, cout)
    b2 = bias.reshape(1, cout)

    def body(x_ref, top_ref, bot_ref, w_ref, b_ref, o_ref):
        win = jnp.concatenate([top_ref[0, 0], x_ref[0], bot_ref[0, 0]], axis=0)
        zc = jnp.zeros((th + 2, 1, cin), win.dtype)
        win = jnp.concatenate([zc, win, zc], axis=1)
        if im2col:
            cols = jnp.concatenate(
                [win[dy:dy + th, dx:dx + wd, :]
                 for dy in range(3) for dx in range(3)], axis=-1)
            res = jnp.dot(cols.reshape(th * wd, 9 * cin), w_ref[...],
                          precision=prec, preferred_element_type=_F32)
        else:
            res = jnp.zeros((th * wd, cout), _F32)
            for dy in range(3):
                for dx in range(3):
                    sl = win[dy:dy + th, dx:dx + wd, :].reshape(th * wd, cin)
                    res = res + jnp.dot(sl, w_ref[dy * 3 + dx],
                                        precision=prec,
                                        preferred_element_type=_F32)
        res = act(res + b_ref[0])
        o_ref[0] = res.reshape(th, wd, cout).astype(out_dtype)

    return pl.pallas_call(
        body,
        grid=(b, t),
        in_specs=[
            pl.BlockSpec((1, th, wd, cin), lambda i, j: (i, j, 0, 0)),
            pl.BlockSpec((1, 1, 1, wd, cin), lambda i, j: (i, j, 0, 0, 0)),
            pl.BlockSpec((1, 1, 1, wd, cin), lambda i, j: (i, j, 0, 0, 0)),
            pl.BlockSpec(wk.shape, lambda i, j: (0,) * wk.ndim),
            pl.BlockSpec((1, cout), lambda i, j: (0, 0)),
        ],
        out_specs=pl.BlockSpec((1, th, wd, cout), lambda i, j: (i, j, 0, 0)),
        out_shape=jax.ShapeDtypeStruct((b, h, wd, cout), out_dtype),
        compiler_params=pltpu.CompilerParams(
            dimension_semantics=("arbitrary", "arbitrary"),
            vmem_limit_bytes=100 * 1024 * 1024,
        ),
    )(x, top, bot, wk, b2)


_PB = 2048


def _vq_argmin(enc2d, codebook):
    n, d = enc2d.shape
    k = codebook.shape[0]
    nblk = n // _PB

    def body(x_ref, cb_ref, idx_ref, loss_ref):
        x = x_ref[...]
        cb = cb_ref[...]
        toks = jnp.swapaxes(x.reshape(_PB // 64, 64, 64), 1, 2).reshape(_PB, 64)
        bf = jnp.bfloat16
        dims = (((1,), (1,)), ((), ()))
        t_hi = toks.astype(bf)
        t_lo = (toks - t_hi.astype(_F32)).astype(bf)
        c_hi = cb.astype(bf)
        c_lo = (cb - c_hi.astype(_F32)).astype(bf)
        mm = (lax.dot_general(t_hi, c_hi, dims, preferred_element_type=_F32)
              + (lax.dot_general(t_hi, c_lo, dims, preferred_element_type=_F32)
                 + lax.dot_general(t_lo, c_hi, dims,
                                   preferred_element_type=_F32)))
        xn = jnp.sum(toks * toks, axis=1)[:, None]
        cn = jnp.sum(cb * cb, axis=1)[None, :]
        dist = xn + cn - 2.0 * mm
        m = jnp.min(dist, axis=1, keepdims=True)
        iota = lax.broadcasted_iota(jnp.int32, (_PB, k), 1)
        idx = jnp.min(jnp.where(dist == m, iota, k), axis=1)
        idx_ref[0, 0] = idx

        @pl.when(pl.program_id(0) == 0)
        def _():
            loss_ref[...] = jnp.zeros((1, 1), _F32)

        loss_ref[...] += jnp.sum(m).reshape(1, 1)

    return pl.pallas_call(
        body,
        grid=(nblk,),
        in_specs=[
            pl.BlockSpec((_PB, d), lambda i: (i, 0)),
            pl.BlockSpec((k, d), lambda i: (0, 0)),
        ],
        out_specs=[
            pl.BlockSpec((1, 1, _PB), lambda i: (i, 0, 0)),
            pl.BlockSpec((1, 1), lambda i: (0, 0)),
        ],
        out_shape=[
            jax.ShapeDtypeStruct((nblk, 1, _PB), jnp.int32),
            jax.ShapeDtypeStruct((1, 1), _F32),
        ],
        compiler_params=pltpu.CompilerParams(
            dimension_semantics=("arbitrary",),
            vmem_limit_bytes=100 * 1024 * 1024,
        ),
    )(enc2d, codebook)


_NW = 32
_CHT = 392
_NCHK = 16


def _sc_gather(codebook, idx_flat):
    n = idx_flat.shape[0]
    d = codebook.shape[1]
    kk = codebook.shape[0]
    per_w = n // _NW
    mesh = plsc.VectorSubcoreMesh(core_axis_name="c", subcore_axis_name="s")

    @functools.partial(
        pl.kernel, mesh=mesh,
        compiler_params=pltpu.CompilerParams(use_tc_tiling_on_sc=False),
        out_type=jax.ShapeDtypeStruct((n * d,), jnp.bfloat16),
        scratch_types=[
            pltpu.VMEM((kk * d,), jnp.bfloat16),
            pltpu.VMEM((per_w + 8,), jnp.int32),
            pltpu.VMEM((2, _CHT * d), jnp.bfloat16),
            pltpu.SemaphoreType.DMA,
            pltpu.SemaphoreType.DMA,
            pltpu.SemaphoreType.DMA,
        ],
    )
    def k(cb_hbm, idx_hbm, out_hbm, cb_v, idx_v, stg, lsem, wsem0, wsem1):
        wid = lax.axis_index("s") * 2 + lax.axis_index("c")
        base = wid * per_w
        pltpu.async_copy(cb_hbm, cb_v, lsem)
        pltpu.async_copy(idx_hbm.at[pl.ds(base, per_w)],
                         idx_v.at[pl.ds(0, per_w)], lsem).wait()
        pltpu.make_async_copy(cb_hbm, cb_v, lsem).wait()

        def pair(p, carry):
            for par in range(2):
                wsem = (wsem0, wsem1)[par]
                c = 2 * p + par

                @pl.when(p > 0)
                def _():
                    pltpu.make_async_copy(
                        stg.at[par], out_hbm.at[pl.ds(0, _CHT * d)],
                        wsem).wait()

                def token8(i8, carry2):
                    vec = idx_v[pl.ds(c * _CHT + i8 * 8, 16)]
                    for l in range(8):
                        s = vec[l]
                        t = i8 * 8 + l
                        for q in range(2):
                            stg[par, pl.ds(t * d + q * 32, 32)] = (
                                cb_v[pl.ds(s * d + q * 32, 32)])
                    return carry2

                lax.fori_loop(0, _CHT // 8, token8, 0)
                pltpu.async_copy(
                    stg.at[par],
                    out_hbm.at[pl.ds((base + c * _CHT) * d, _CHT * d)], wsem)
            return carry

        lax.fori_loop(0, _NCHK // 2, pair, 0)
        for par in range(2):
            wsem = (wsem0, wsem1)[par]
            pltpu.make_async_copy(stg.at[par],
                                  out_hbm.at[pl.ds(0, _CHT * d)], wsem).wait()

    return k(codebook.astype(jnp.bfloat16).reshape(-1), idx_flat).reshape(n, d)


def kernel(x, enc_w1, enc_b1, enc_w2, enc_b2, codebook,
           dec_w1, dec_b1, dec_w2, dec_b2):
    b, _, h, w = x.shape
    d = codebook.shape[1]
    relu = lambda v: jnp.maximum(v, 0.0)

    x_nhwc = jnp.transpose(x, (0, 2, 3, 1))
    h1 = _conv3x3(x_nhwc, enc_w1, enc_b1, 16, relu, im2col=True,
                  out_dtype=jnp.bfloat16)
    enc = _conv3x3(h1, enc_w2, enc_b2, 8, relu, im2col=False)

    enc2d = enc.reshape(b * h * w, d)
    idx_blocks, loss_sum = _vq_argmin(enc2d, codebook)
    n_tok = b * h * w
    groups = h * w // d
    idx_flat = (idx_blocks.reshape(b, groups, d)
                .transpose(0, 2, 1).reshape(n_tok))

    quant = _sc_gather(codebook, idx_flat)
    q_nhwc = jnp.transpose(quant.reshape(b, d, h, w), (0, 2, 3, 1))

    d1 = _conv3x3(q_nhwc, dec_w1, dec_b1, 8, relu, im2col=False,
                  out_dtype=jnp.bfloat16)
    out_nhwc = _conv3x3(d1, dec_w2, dec_b2, 8, jax.nn.sigmoid, im2col=False)
    reconstructed = jnp.transpose(out_nhwc, (0, 3, 1, 2))

    loss = loss_sum[0, 0] * (1.25 / (n_tok * d))
    return (reconstructed, loss)
